# baseline (speedup 1.0000x reference)
import functools

import jax
import jax.numpy as jnp
from jax import lax
from jax.experimental import pallas as pl
from jax.experimental.pallas import tpu as pltpu
from jax.experimental.pallas import tpu_sc as plsc

_B = 16384
_LQ = 20
_LT = 200
_V = 1000
_D = 64
_H = 64

_NC = 2
_NS = 16
_NW = _NC * _NS
_BPW = _B // _NW
_CB = 128
_NCHUNK = _BPW // _CB


def _sc_pool_body(qt_hbm, tt_hbm, emb_hbm, qsum_hbm, tsum_hbm,
                  table_v, qtok_v, ttok_v, qs_v, ts_v, in_sem, out_sem):
    c = lax.axis_index("c")
    s = lax.axis_index("s")
    wid = s * _NC + c
    base = wid * _BPW

    pltpu.sync_copy(emb_hbm, table_v)

    lane = lax.iota(jnp.int32, 16)
    hi_mask = jnp.full((16,), 0xFFFF0000, jnp.uint32)
    zero32 = jnp.zeros((32,), jnp.bfloat16)

    def pooled16(tv, accs):
        p0, p1 = zero32, zero32
        for i in range(16):
            t = tv[i]
            p0 = p0 + table_v[t, pl.ds(0, 32)]
            p1 = p1 + table_v[t, pl.ds(32, 32)]
        a0, a1, a2, a3 = accs
        r0 = plsc.bitcast(p0, jnp.uint32)
        r1 = plsc.bitcast(p1, jnp.uint32)
        a0 = a0 + plsc.bitcast(r0 << 16, jnp.float32)
        a1 = a1 + plsc.bitcast(r0 & hi_mask, jnp.float32)
        a2 = a2 + plsc.bitcast(r1 << 16, jnp.float32)
        a3 = a3 + plsc.bitcast(r1 & hi_mask, jnp.float32)
        return a0, a1, a2, a3

    def accum_chunk(tok_ref, e, nfull, tail_off, keep_from, keep_to, out_ref):
        zero = jnp.zeros((16,), jnp.float32)

        def vreg_chunk(r, accs):
            return pooled16(tok_ref[e, pl.ds(r * 16, 16)], accs)

        accs = lax.fori_loop(0, nfull, vreg_chunk, (zero,) * 4)
        tv = tok_ref[e, pl.ds(tail_off, 16)]
        tv = jnp.where((lane >= keep_from) & (lane < keep_to), tv, 0)
        accs = pooled16(tv, accs)
        for j in range(4):
            out_ref[e, pl.ds(16 * j, 16)] = accs[j]

    def in_start(k, buf):
        cb = base + k * _CB
        pltpu.make_async_copy(qt_hbm.at[pl.ds(cb, _CB)], qtok_v.at[buf],
                              in_sem).start()
        pltpu.make_async_copy(tt_hbm.at[pl.ds(cb, _CB)], ttok_v.at[buf],
                              in_sem).start()

    def in_wait(buf):
        pltpu.make_async_copy(qt_hbm.at[pl.ds(base, _CB)], qtok_v.at[buf],
                              in_sem).wait()
        pltpu.make_async_copy(tt_hbm.at[pl.ds(base, _CB)], ttok_v.at[buf],
                              in_sem).wait()

    def out_start(k, buf):
        cb = base + k * _CB
        pltpu.make_async_copy(qs_v.at[buf], qsum_hbm.at[pl.ds(cb, _CB)],
                              out_sem).start()
        pltpu.make_async_copy(ts_v.at[buf], tsum_hbm.at[pl.ds(cb, _CB)],
                              out_sem).start()

    def out_wait(buf):
        pltpu.make_async_copy(qs_v.at[buf], qsum_hbm.at[pl.ds(base, _CB)],
                              out_sem).wait()
        pltpu.make_async_copy(ts_v.at[buf], tsum_hbm.at[pl.ds(base, _CB)],
                              out_sem).wait()

    in_start(0, 0)
    for k in range(_NCHUNK):
        buf = k % 2
        in_wait(buf)
        if k + 1 < _NCHUNK:
            in_start(k + 1, 1 - buf)
        if k >= 2:
            out_wait(buf)

        def ex_body(e, _, buf=buf):
            accum_chunk(qtok_v.at[buf], e, 1, 4, 12, 16, qs_v.at[buf])
            accum_chunk(ttok_v.at[buf], e, 12, 184, 8, 16, ts_v.at[buf])
            return 0

        lax.fori_loop(0, _CB, ex_body, 0)
        out_start(k, buf)
    out_wait(_NCHUNK % 2)
    out_wait(1 - _NCHUNK % 2)


_PERM = ([2 * k for k in range(16)] + [2 * k + 1 for k in range(16)]
         + [32 + 2 * k for k in range(16)] + [33 + 2 * k for k in range(16)])


@jax.jit
def _sc_pool(query_tokens, title_tokens, emb):
    emb16 = emb.astype(jnp.bfloat16)
    mesh = plsc.VectorSubcoreMesh(core_axis_name="c", subcore_axis_name="s",
                                  num_cores=_NC, num_subcores=_NS)
    f = pl.kernel(
        _sc_pool_body,
        out_type=[jax.ShapeDtypeStruct((_B, _D), jnp.float32),
                  jax.ShapeDtypeStruct((_B, _D), jnp.float32)],
        mesh=mesh,
        scratch_types=[
            pltpu.VMEM((_V, _D), jnp.bfloat16),
            pltpu.VMEM((2, _CB, _LQ), jnp.int32),
            pltpu.VMEM((2, _CB, _LT), jnp.int32),
            pltpu.VMEM((2, _CB, _D), jnp.float32),
            pltpu.VMEM((2, _CB, _D), jnp.float32),
            pltpu.SemaphoreType.DMA,
            pltpu.SemaphoreType.DMA,
        ],
        compiler_params=pltpu.CompilerParams(use_tc_tiling_on_sc=False, needs_layout_passes=False),
    )
    return f(query_tokens, title_tokens, emb16)


_TC_BLK = 512


def _tc_head_body(qs_ref, ts_ref, wq_ref, bq_ref, wt_ref, bt_ref, out_ref):
    qh = jnp.tanh(
        lax.dot_general(qs_ref[...], wq_ref[...], (((1,), (1,)), ((), ())),
                        preferred_element_type=jnp.float32) + bq_ref[...])
    th = jnp.tanh(
        lax.dot_general(ts_ref[...], wt_ref[...], (((1,), (1,)), ((), ())),
                        preferred_element_type=jnp.float32) + bt_ref[...])
    w12 = jnp.sum(qh * th, axis=1, keepdims=True)
    w1s = jnp.sum(qh * qh, axis=1, keepdims=True)
    w2s = jnp.sum(th * th, axis=1, keepdims=True)
    cos = w12 / (jnp.sqrt(w1s) * jnp.sqrt(w2s) + 1e-12)
    out_ref[...] = (cos + 1.0) * 0.5


@jax.jit
def _tc_head(qsum, tsum, Wq, bq, Wt, bt):
    grid = (_B // _TC_BLK,)
    return pl.pallas_call(
        _tc_head_body,
        grid=grid,
        in_specs=[
            pl.BlockSpec((_TC_BLK, _D), lambda i: (i, 0)),
            pl.BlockSpec((_TC_BLK, _D), lambda i: (i, 0)),
            pl.BlockSpec((_H, _D), lambda i: (0, 0)),
            pl.BlockSpec((1, _H), lambda i: (0, 0)),
            pl.BlockSpec((_H, _D), lambda i: (0, 0)),
            pl.BlockSpec((1, _H), lambda i: (0, 0)),
        ],
        out_specs=pl.BlockSpec((_TC_BLK, 1), lambda i: (i, 0)),
        out_shape=jax.ShapeDtypeStruct((_B, 1), jnp.float32),
    )(qsum, tsum, Wq, bq.reshape(1, _H), Wt, bt.reshape(1, _H))


def kernel(query_tokens, title_tokens, emb, Wq, bq, Wt, bt):
    qsum, tsum = _sc_pool(query_tokens, title_tokens, emb)
    perm = jnp.asarray(_PERM, jnp.int32)
    return _tc_head(qsum, tsum, Wq[:, perm], bq, Wt[:, perm], bt)


# speedup vs baseline: 1.0771x; 1.0771x over previous
import functools

import jax
import jax.numpy as jnp
from jax import lax
from jax.experimental import pallas as pl
from jax.experimental.pallas import tpu as pltpu
from jax.experimental.pallas import tpu_sc as plsc

_B = 16384
_LQ = 20
_LT = 200
_V = 1000
_D = 64
_H = 64

_NC = 2
_NS = 16
_NW = _NC * _NS
_BPW = _B // _NW
_CB = 128
_NCHUNK = _BPW // _CB


def _sc_pool_body(qt_hbm, tt_hbm, emb_hbm, qsum_hbm, tsum_hbm,
                  table_v, qtok_v, ttok_v, qs_v, ts_v, in_sem, out_sem):
    c = lax.axis_index("c")
    s = lax.axis_index("s")
    wid = s * _NC + c
    base = wid * _BPW

    pltpu.sync_copy(emb_hbm, table_v)

    lane = lax.iota(jnp.int32, 16)
    hi_mask = jnp.full((16,), 0xFFFF0000, jnp.uint32)
    zero32 = jnp.zeros((32,), jnp.bfloat16)

    def pooled16(tv, accs):
        p0, p1 = zero32, zero32
        for i in range(16):
            t = tv[i]
            p0 = p0 + table_v[t, pl.ds(0, 32)]
            p1 = p1 + table_v[t, pl.ds(32, 32)]
        a0, a1, a2, a3 = accs
        r0 = plsc.bitcast(p0, jnp.uint32)
        r1 = plsc.bitcast(p1, jnp.uint32)
        a0 = a0 + plsc.bitcast(r0 << 16, jnp.float32)
        a1 = a1 + plsc.bitcast(r0 & hi_mask, jnp.float32)
        a2 = a2 + plsc.bitcast(r1 << 16, jnp.float32)
        a3 = a3 + plsc.bitcast(r1 & hi_mask, jnp.float32)
        return a0, a1, a2, a3

    def accum_chunk(tok_ref, e, nfull, tail_off, keep_from, keep_to, out_ref):
        zero = jnp.zeros((16,), jnp.float32)

        def vreg_chunk(r, accs):
            return pooled16(tok_ref[e, pl.ds(r * 16, 16)], accs)

        accs = lax.fori_loop(0, nfull, vreg_chunk, (zero,) * 4)
        tv = tok_ref[e, pl.ds(tail_off, 16)]
        tv = jnp.where((lane >= keep_from) & (lane < keep_to), tv, 0)
        accs = pooled16(tv, accs)
        half = (e & 1) * 64
        for j in range(4):
            out_ref[e >> 1, pl.ds(half + 16 * j, 16)] = accs[j]

    def in_start(k, buf):
        cb = base + k * _CB
        pltpu.make_async_copy(qt_hbm.at[pl.ds(cb, _CB)], qtok_v.at[buf],
                              in_sem).start()
        pltpu.make_async_copy(tt_hbm.at[pl.ds(cb, _CB)], ttok_v.at[buf],
                              in_sem).start()

    def in_wait(buf):
        pltpu.make_async_copy(qt_hbm.at[pl.ds(base, _CB)], qtok_v.at[buf],
                              in_sem).wait()
        pltpu.make_async_copy(tt_hbm.at[pl.ds(base, _CB)], ttok_v.at[buf],
                              in_sem).wait()

    def out_start(k, buf):
        cb = (base + k * _CB) // 2
        pltpu.make_async_copy(qs_v.at[buf], qsum_hbm.at[pl.ds(cb, _CB // 2)],
                              out_sem).start()
        pltpu.make_async_copy(ts_v.at[buf], tsum_hbm.at[pl.ds(cb, _CB // 2)],
                              out_sem).start()

    def out_wait(buf):
        pltpu.make_async_copy(qs_v.at[buf],
                              qsum_hbm.at[pl.ds(base // 2, _CB // 2)],
                              out_sem).wait()
        pltpu.make_async_copy(ts_v.at[buf],
                              tsum_hbm.at[pl.ds(base // 2, _CB // 2)],
                              out_sem).wait()

    in_start(0, 0)
    for k in range(_NCHUNK):
        buf = k % 2
        in_wait(buf)
        if k + 1 < _NCHUNK:
            in_start(k + 1, 1 - buf)
        if k >= 2:
            out_wait(buf)

        def ex_body(e, _, buf=buf):
            accum_chunk(qtok_v.at[buf], e, 1, 4, 12, 16, qs_v.at[buf])
            accum_chunk(ttok_v.at[buf], e, 12, 184, 8, 16, ts_v.at[buf])
            return 0

        lax.fori_loop(0, _CB, ex_body, 0)
        out_start(k, buf)
    out_wait(_NCHUNK % 2)
    out_wait(1 - _NCHUNK % 2)


_PERM = ([2 * k for k in range(16)] + [2 * k + 1 for k in range(16)]
         + [32 + 2 * k for k in range(16)] + [33 + 2 * k for k in range(16)])


@jax.jit
def _sc_pool(query_tokens, title_tokens, emb):
    emb16 = emb.astype(jnp.bfloat16)
    mesh = plsc.VectorSubcoreMesh(core_axis_name="c", subcore_axis_name="s",
                                  num_cores=_NC, num_subcores=_NS)
    f = pl.kernel(
        _sc_pool_body,
        out_type=[jax.ShapeDtypeStruct((_B // 2, 2 * _D), jnp.float32),
                  jax.ShapeDtypeStruct((_B // 2, 2 * _D), jnp.float32)],
        mesh=mesh,
        scratch_types=[
            pltpu.VMEM((_V, _D), jnp.bfloat16),
            pltpu.VMEM((2, _CB, _LQ), jnp.int32),
            pltpu.VMEM((2, _CB, _LT), jnp.int32),
            pltpu.VMEM((2, _CB // 2, 2 * _D), jnp.float32),
            pltpu.VMEM((2, _CB // 2, 2 * _D), jnp.float32),
            pltpu.SemaphoreType.DMA,
            pltpu.SemaphoreType.DMA,
        ],
        compiler_params=pltpu.CompilerParams(use_tc_tiling_on_sc=False, needs_layout_passes=False),
    )
    return f(query_tokens, title_tokens, emb16)


_TC_BLK = 512


def _tc_head_body(qs_ref, ts_ref, wq_ref, bq_ref, wt_ref, bt_ref, out_ref):
    qh = jnp.tanh(
        lax.dot_general(qs_ref[...], wq_ref[...], (((1,), (0,)), ((), ())),
                        preferred_element_type=jnp.float32) + bq_ref[...])
    th = jnp.tanh(
        lax.dot_general(ts_ref[...], wt_ref[...], (((1,), (0,)), ((), ())),
                        preferred_element_type=jnp.float32) + bt_ref[...])
    prod = qh * th
    qq = qh * qh
    tt = th * th
    for h in range(2):
        sl = slice(64 * h, 64 * (h + 1))
        w12 = jnp.sum(prod[:, sl], axis=1, keepdims=True)
        w1s = jnp.sum(qq[:, sl], axis=1, keepdims=True)
        w2s = jnp.sum(tt[:, sl], axis=1, keepdims=True)
        cos = w12 / (jnp.sqrt(w1s) * jnp.sqrt(w2s) + 1e-12)
        out_ref[:, h:h + 1] = (cos + 1.0) * 0.5


@jax.jit
def _tc_head(qsum, tsum, W2q, b2q, W2t, b2t):
    grid = (_B // 2 // _TC_BLK,)
    return pl.pallas_call(
        _tc_head_body,
        grid=grid,
        in_specs=[
            pl.BlockSpec((_TC_BLK, 2 * _D), lambda i: (i, 0)),
            pl.BlockSpec((_TC_BLK, 2 * _D), lambda i: (i, 0)),
            pl.BlockSpec((2 * _H, 2 * _H), lambda i: (0, 0)),
            pl.BlockSpec((1, 2 * _H), lambda i: (0, 0)),
            pl.BlockSpec((2 * _H, 2 * _H), lambda i: (0, 0)),
            pl.BlockSpec((1, 2 * _H), lambda i: (0, 0)),
        ],
        out_specs=pl.BlockSpec((_TC_BLK, 2), lambda i: (i, 0)),
        out_shape=jax.ShapeDtypeStruct((_B // 2, 2), jnp.float32),
    )(qsum, tsum, W2q, b2q.reshape(1, 2 * _H), W2t, b2t.reshape(1, 2 * _H))


def _blockdiag2(W):
    z = jnp.zeros((_H, _H), W.dtype)
    return jnp.concatenate(
        [jnp.concatenate([W, z], axis=1),
         jnp.concatenate([z, W], axis=1)], axis=0)


def kernel(query_tokens, title_tokens, emb, Wq, bq, Wt, bt):
    qsum, tsum = _sc_pool(query_tokens, title_tokens, emb)
    perm = jnp.asarray(_PERM, jnp.int32)
    W2q = _blockdiag2(Wq[:, perm].T)
    W2t = _blockdiag2(Wt[:, perm].T)
    b2q = jnp.concatenate([bq, bq])
    b2t = jnp.concatenate([bt, bt])
    out2 = _tc_head(qsum, tsum, W2q, b2q, W2t, b2t)
    return out2.reshape(_B, 1)


# speedup vs baseline: 1.0944x; 1.0160x over previous
import functools

import jax
import jax.numpy as jnp
from jax import lax
from jax.experimental import pallas as pl
from jax.experimental.pallas import tpu as pltpu
from jax.experimental.pallas import tpu_sc as plsc

_B = 16384
_LQ = 20
_LT = 200
_V = 1000
_D = 64
_H = 64

_NC = 2
_NS = 16
_NW = _NC * _NS
_NSLICE = 4
_BS = _B // _NSLICE
_BPW = _BS // _NW
_CB = 128
_NCHUNK = _BPW // _CB


def _sc_pool_body(qt_hbm, tt_hbm, emb_hbm, qsum_hbm, tsum_hbm,
                  table_v, qtok_v, ttok_v, qs_v, ts_v, in_sem, out_sem):
    c = lax.axis_index("c")
    s = lax.axis_index("s")
    wid = s * _NC + c
    base = wid * _BPW

    pltpu.sync_copy(emb_hbm, table_v)

    lane = lax.iota(jnp.int32, 16)
    hi_mask = jnp.full((16,), 0xFFFF0000, jnp.uint32)
    zero32 = jnp.zeros((32,), jnp.bfloat16)

    def pooled16(tv, accs):
        p0, p1 = zero32, zero32
        for i in range(16):
            t = tv[i]
            p0 = p0 + table_v[t, pl.ds(0, 32)]
            p1 = p1 + table_v[t, pl.ds(32, 32)]
        a0, a1, a2, a3 = accs
        r0 = plsc.bitcast(p0, jnp.uint32)
        r1 = plsc.bitcast(p1, jnp.uint32)
        a0 = a0 + plsc.bitcast(r0 << 16, jnp.float32)
        a1 = a1 + plsc.bitcast(r0 & hi_mask, jnp.float32)
        a2 = a2 + plsc.bitcast(r1 << 16, jnp.float32)
        a3 = a3 + plsc.bitcast(r1 & hi_mask, jnp.float32)
        return a0, a1, a2, a3

    def accum_chunk(tok_ref, e, nfull, tail_off, keep_from, keep_to, out_ref):
        zero = jnp.zeros((16,), jnp.float32)

        def vreg_chunk(r, accs):
            return pooled16(tok_ref[e, pl.ds(r * 16, 16)], accs)

        accs = lax.fori_loop(0, nfull, vreg_chunk, (zero,) * 4)
        tv = tok_ref[e, pl.ds(tail_off, 16)]
        tv = jnp.where((lane >= keep_from) & (lane < keep_to), tv, 0)
        accs = pooled16(tv, accs)
        half = (e & 1) * 64
        for j in range(4):
            out_ref[e >> 1, pl.ds(half + 16 * j, 16)] = accs[j]

    def in_start(k, buf):
        cb = base + k * _CB
        pltpu.make_async_copy(qt_hbm.at[pl.ds(cb, _CB)], qtok_v.at[buf],
                              in_sem).start()
        pltpu.make_async_copy(tt_hbm.at[pl.ds(cb, _CB)], ttok_v.at[buf],
                              in_sem).start()

    def in_wait(buf):
        pltpu.make_async_copy(qt_hbm.at[pl.ds(base, _CB)], qtok_v.at[buf],
                              in_sem).wait()
        pltpu.make_async_copy(tt_hbm.at[pl.ds(base, _CB)], ttok_v.at[buf],
                              in_sem).wait()

    def out_start(k, buf):
        cb = (base + k * _CB) // 2
        pltpu.make_async_copy(qs_v.at[buf], qsum_hbm.at[pl.ds(cb, _CB // 2)],
                              out_sem).start()
        pltpu.make_async_copy(ts_v.at[buf], tsum_hbm.at[pl.ds(cb, _CB // 2)],
                              out_sem).start()

    def out_wait(buf):
        pltpu.make_async_copy(qs_v.at[buf],
                              qsum_hbm.at[pl.ds(base // 2, _CB // 2)],
                              out_sem).wait()
        pltpu.make_async_copy(ts_v.at[buf],
                              tsum_hbm.at[pl.ds(base // 2, _CB // 2)],
                              out_sem).wait()

    in_start(0, 0)
    for k in range(_NCHUNK):
        buf = k % 2
        in_wait(buf)
        if k + 1 < _NCHUNK:
            in_start(k + 1, 1 - buf)
        if k >= 2:
            out_wait(buf)

        def ex_body(e, _, buf=buf):
            accum_chunk(qtok_v.at[buf], e, 1, 4, 12, 16, qs_v.at[buf])
            accum_chunk(ttok_v.at[buf], e, 12, 184, 8, 16, ts_v.at[buf])
            return 0

        lax.fori_loop(0, _CB, ex_body, 0)
        out_start(k, buf)
    for b in range(min(_NCHUNK, 2)):
        out_wait((_NCHUNK - 1 - b) % 2)


_PERM = ([2 * k for k in range(16)] + [2 * k + 1 for k in range(16)]
         + [32 + 2 * k for k in range(16)] + [33 + 2 * k for k in range(16)])


@jax.jit
def _sc_pool(query_tokens, title_tokens, emb16):
    mesh = plsc.VectorSubcoreMesh(core_axis_name="c", subcore_axis_name="s",
                                  num_cores=_NC, num_subcores=_NS)
    f = pl.kernel(
        _sc_pool_body,
        out_type=[jax.ShapeDtypeStruct((_BS // 2, 2 * _D), jnp.float32),
                  jax.ShapeDtypeStruct((_BS // 2, 2 * _D), jnp.float32)],
        mesh=mesh,
        scratch_types=[
            pltpu.VMEM((_V, _D), jnp.bfloat16),
            pltpu.VMEM((2, _CB, _LQ), jnp.int32),
            pltpu.VMEM((2, _CB, _LT), jnp.int32),
            pltpu.VMEM((2, _CB // 2, 2 * _D), jnp.float32),
            pltpu.VMEM((2, _CB // 2, 2 * _D), jnp.float32),
            pltpu.SemaphoreType.DMA,
            pltpu.SemaphoreType.DMA,
        ],
        compiler_params=pltpu.CompilerParams(use_tc_tiling_on_sc=False, needs_layout_passes=False),
    )
    return f(query_tokens, title_tokens, emb16)


_TC_BLK = 512


def _tc_head_body(qs_ref, ts_ref, wq_ref, bq_ref, wt_ref, bt_ref, out_ref):
    qh = jnp.tanh(
        lax.dot_general(qs_ref[...], wq_ref[...], (((1,), (0,)), ((), ())),
                        preferred_element_type=jnp.float32) + bq_ref[...])
    th = jnp.tanh(
        lax.dot_general(ts_ref[...], wt_ref[...], (((1,), (0,)), ((), ())),
                        preferred_element_type=jnp.float32) + bt_ref[...])
    prod = qh * th
    qq = qh * qh
    tt = th * th
    for h in range(2):
        sl = slice(64 * h, 64 * (h + 1))
        w12 = jnp.sum(prod[:, sl], axis=1, keepdims=True)
        w1s = jnp.sum(qq[:, sl], axis=1, keepdims=True)
        w2s = jnp.sum(tt[:, sl], axis=1, keepdims=True)
        cos = w12 / (jnp.sqrt(w1s) * jnp.sqrt(w2s) + 1e-12)
        out_ref[:, h:h + 1] = (cos + 1.0) * 0.5


@jax.jit
def _tc_head(qsum, tsum, W2q, b2q, W2t, b2t):
    grid = (_BS // 2 // _TC_BLK,)
    return pl.pallas_call(
        _tc_head_body,
        grid=grid,
        in_specs=[
            pl.BlockSpec((_TC_BLK, 2 * _D), lambda i: (i, 0)),
            pl.BlockSpec((_TC_BLK, 2 * _D), lambda i: (i, 0)),
            pl.BlockSpec((2 * _H, 2 * _H), lambda i: (0, 0)),
            pl.BlockSpec((1, 2 * _H), lambda i: (0, 0)),
            pl.BlockSpec((2 * _H, 2 * _H), lambda i: (0, 0)),
            pl.BlockSpec((1, 2 * _H), lambda i: (0, 0)),
        ],
        out_specs=pl.BlockSpec((_TC_BLK, 2), lambda i: (i, 0)),
        out_shape=jax.ShapeDtypeStruct((_BS // 2, 2), jnp.float32),
    )(qsum, tsum, W2q, b2q.reshape(1, 2 * _H), W2t, b2t.reshape(1, 2 * _H))


def _blockdiag2(W):
    z = jnp.zeros((_H, _H), W.dtype)
    return jnp.concatenate(
        [jnp.concatenate([W, z], axis=1),
         jnp.concatenate([z, W], axis=1)], axis=0)


def kernel(query_tokens, title_tokens, emb, Wq, bq, Wt, bt):
    perm = jnp.asarray(_PERM, jnp.int32)
    W2q = _blockdiag2(Wq[:, perm].T)
    W2t = _blockdiag2(Wt[:, perm].T)
    b2q = jnp.concatenate([bq, bq])
    b2t = jnp.concatenate([bt, bt])
    emb16 = emb.astype(jnp.bfloat16)
    outs = []
    for i in range(_NSLICE):
        sl = slice(i * _BS, (i + 1) * _BS)
        qsum, tsum = _sc_pool(query_tokens[sl], title_tokens[sl], emb16)
        outs.append(_tc_head(qsum, tsum, W2q, b2q, W2t, b2t))
    return jnp.concatenate(outs, axis=0).reshape(_B, 1)
